# baseline (speedup 1.0000x reference)
import functools

import jax
import jax.numpy as jnp
from jax import lax
from jax.experimental import pallas as pl
from jax.experimental.pallas import tpu as pltpu


def _build_body(num_enc, num_dec, T, future_len, H, W, CP, S, NQ, OC8):
    L = num_enc + num_dec
    PW = W + 2
    NR = H * PW
    C2 = 2 * CP
    G4 = 4 * CP
    K9 = 9 * C2
    deltas = [0, -PW - 1, -PW, -PW + 1, -1, 1, PW - 1, PW, PW + 1]

    def body(x_ref, w_ref, b_ref, o_ref, h_ref, c_ref, q_ref):
        h_ref[...] = jnp.zeros_like(h_ref)
        c_ref[...] = jnp.zeros_like(c_ref)
        q_ref[...] = jnp.zeros_like(q_ref)

        lane = lax.broadcasted_iota(jnp.int32, (1, G4), 1)
        sig = lane < 3 * CP
        sc = jnp.where(sig, 0.5, 1.0)
        sb = jnp.where(sig, 0.5, 0.0)
        rr = lax.broadcasted_iota(jnp.int32, (NR, 1), 0) % PW
        col_ok = jnp.logical_and(rr != 0, rr != PW - 1)

        def cell(layer, inp_bf16):
            wi = inp_bf16.shape[-1]
            q_ref[S:S + NR, 0:wi] = inp_bf16
            if wi < CP:
                q_ref[S:S + NR, wi:CP] = jnp.zeros((NR, CP - wi),
                                                   jnp.bfloat16)
            q_ref[S:S + NR, CP:C2] = h_ref[layer].astype(jnp.bfloat16)
            for gi in range(1, 9):
                d = deltas[gi]
                q_ref[S:S + NR, C2 * gi:C2 * (gi + 1)] = (
                    q_ref[S + d:S + d + NR, 0:C2])
            gates = jnp.dot(q_ref[S:S + NR, :], w_ref[layer],
                            preferred_element_type=jnp.float32) + b_ref[layer]
            act = jnp.tanh(gates) * sc + sb
            i_g = act[:, 0 * CP:1 * CP]
            f_g = act[:, 1 * CP:2 * CP]
            o_g = act[:, 2 * CP:3 * CP]
            g_g = act[:, 3 * CP:4 * CP]
            c_n = f_g * c_ref[layer] + i_g * g_g
            h_ref[layer] = jnp.where(col_ok, o_g * jnp.tanh(c_n), 0.0)
            c_ref[layer] = c_n

        def enc_step(t, carry):
            frame = x_ref[0, t].reshape(NR, x_ref.shape[-1])
            cell(0, frame)
            for i in range(1, num_enc):
                cell(i, h_ref[i - 1].astype(jnp.bfloat16))
            return carry

        lax.fori_loop(0, T, enc_step, 0)

        def dec_step(j, carry):
            cell(num_enc, h_ref[num_enc - 1].astype(jnp.bfloat16))
            for d in range(1, num_dec):
                layer = num_enc + d
                cell(layer, h_ref[layer - 1].astype(jnp.bfloat16))
            o_ref[0, j] = h_ref[L - 1][:, 0:OC8].reshape(H, PW, OC8)
            return carry

        lax.fori_loop(0, future_len, dec_step, 0)

    return body


def _pack(layers, CP):
    PW_taps = [(1, 1), (0, 0), (0, 1), (0, 2), (1, 0), (1, 2),
               (2, 0), (2, 1), (2, 2)]
    L = len(layers)
    C2 = 2 * CP
    w_all = jnp.zeros((L, 9 * C2, 4 * CP), jnp.float32)
    b_all = jnp.zeros((L, 1, 4 * CP), jnp.float32)
    for l, (w, b) in enumerate(layers):
        ctot, ch = w.shape[-2], w.shape[-1]
        cin = ctot - ch
        for gi, (ky, kx) in enumerate(PW_taps):
            wt = jnp.transpose(w[:, ky, kx], (1, 0, 2))
            wt = jnp.pad(wt, ((0, 0), (0, 0), (0, CP - ch)))
            wt = wt * jnp.array([0.5, 0.5, 0.5, 1.0]).reshape(1, 4, 1)
            wt = wt.reshape(ctot, 4 * CP)
            r0 = C2 * gi
            w_all = w_all.at[l, r0:r0 + cin, :].set(wt[:cin])
            w_all = w_all.at[l, r0 + CP:r0 + CP + ch, :].set(wt[cin:])
        bb = jnp.pad(b.reshape(4, ch), ((0, 0), (0, CP - ch)))
        bb = (bb * jnp.array([0.5, 0.5, 0.5, 1.0]).reshape(4, 1)).reshape(
            4 * CP)
        b_all = b_all.at[l, 0].set(bb)
    return w_all.astype(jnp.bfloat16), b_all


def _forward(x, enc_params, dec_params, future_len):
    B, c_in, T, H, W = x.shape
    num_enc, num_dec = len(enc_params), len(dec_params)
    L = num_enc + num_dec
    hidden = enc_params[0][0].shape[-1]
    c_out = dec_params[-1][0].shape[-1]
    CP = max(c_in, hidden, c_out)
    CP = ((CP + 31) // 32) * 32
    PW = W + 2
    NR = H * PW
    K9 = 9 * 2 * CP
    IMG0 = (-PW) % 16
    S = IMG0 + PW
    NQ = ((S + NR + PW + 1) + 15) // 16 * 16

    CIN8 = min(CP, ((c_in + 7) // 8) * 8)
    OC8 = min(CP, ((c_out + 7) // 8) * 8)

    w_all, b_all = _pack(list(enc_params) + list(dec_params), CP)

    x_l = jnp.transpose(x.astype(jnp.bfloat16), (0, 2, 3, 4, 1))
    x_l = jnp.pad(x_l, ((0, 0), (0, 0), (0, 0), (1, 1), (0, CIN8 - c_in)))

    body = _build_body(num_enc, num_dec, T, future_len, H, W, CP, S, NQ, OC8)

    out = pl.pallas_call(
        body,
        out_shape=jax.ShapeDtypeStruct((B, future_len, H, PW, OC8),
                                       jnp.float32),
        grid=(B,),
        in_specs=[
            pl.BlockSpec((1, T, H, PW, CIN8), lambda b: (b, 0, 0, 0, 0)),
            pl.BlockSpec((L, K9, 4 * CP), lambda b: (0, 0, 0)),
            pl.BlockSpec((L, 1, 4 * CP), lambda b: (0, 0, 0)),
        ],
        out_specs=pl.BlockSpec((1, future_len, H, PW, OC8),
                               lambda b: (b, 0, 0, 0, 0)),
        scratch_shapes=[
            pltpu.VMEM((L, NR, CP), jnp.float32),
            pltpu.VMEM((L, NR, CP), jnp.float32),
            pltpu.VMEM((NQ, K9), jnp.bfloat16),
        ],
        compiler_params=pltpu.CompilerParams(
            dimension_semantics=("parallel",),
            vmem_limit_bytes=64 * 1024 * 1024),
    )(x_l, w_all, b_all)

    return [jnp.transpose(out[:, j, :, 1:W + 1, :c_out], (0, 3, 1, 2))
            for j in range(future_len)]


def kernel(x, w_e0, b_e0, w_e1, b_e1, w_d0, b_d0, w_d1, b_d1):
    enc_params = [(w_e0, b_e0), (w_e1, b_e1)]
    dec_params = [(w_d0, b_d0), (w_d1, b_d1)]
    return _forward(x, enc_params, dec_params, future_len=10)


# speedup vs baseline: 1.1493x; 1.1493x over previous
import functools

import jax
import jax.numpy as jnp
from jax import lax
from jax.experimental import pallas as pl
from jax.experimental.pallas import tpu as pltpu


def _build_body(num_enc, num_dec, T, future_len, H, W, CP, S, NQ, OC8):
    L = num_enc + num_dec
    PW = W + 2
    NR = H * PW
    C2 = 2 * CP
    G4 = 4 * CP
    K9 = 9 * C2
    deltas = [0, -PW - 1, -PW, -PW + 1, -1, 1, PW - 1, PW, PW + 1]

    def body(x_ref, w_ref, b_ref, o_ref, h_ref, c_ref, q_ref):
        h_ref[...] = jnp.zeros_like(h_ref)
        c_ref[...] = jnp.zeros_like(c_ref)
        q_ref[...] = jnp.zeros_like(q_ref)

        lane = lax.broadcasted_iota(jnp.int32, (1, G4), 1)
        sig = lane < 3 * CP
        sc = jnp.where(sig, 0.5, 1.0)
        sb = jnp.where(sig, 0.5, 0.0)
        rr = lax.broadcasted_iota(jnp.int32, (NR, 1), 0) % PW
        col_ok = jnp.logical_and(rr != 0, rr != PW - 1)

        def cell(e, layer, inp_bf16):
            wi = inp_bf16.shape[-1]
            q_ref[e, S:S + NR, 0:wi] = inp_bf16
            if wi < CP:
                q_ref[e, S:S + NR, wi:CP] = jnp.zeros((NR, CP - wi),
                                                      jnp.bfloat16)
            q_ref[e, S:S + NR, CP:C2] = h_ref[e, layer].astype(jnp.bfloat16)
            for gi in range(1, 9):
                d = deltas[gi]
                q_ref[e, S:S + NR, C2 * gi:C2 * (gi + 1)] = (
                    q_ref[e, S + d:S + d + NR, 0:C2])
            gates = jnp.dot(q_ref[e, S:S + NR, :], w_ref[layer],
                            preferred_element_type=jnp.float32) + b_ref[layer]
            act = jnp.tanh(gates) * sc + sb
            i_g = act[:, 0 * CP:1 * CP]
            f_g = act[:, 1 * CP:2 * CP]
            o_g = act[:, 2 * CP:3 * CP]
            g_g = act[:, 3 * CP:4 * CP]
            c_n = f_g * c_ref[e, layer] + i_g * g_g
            h_ref[e, layer] = jnp.where(col_ok, o_g * jnp.tanh(c_n), 0.0)
            c_ref[e, layer] = c_n

        def enc_step(t, carry):
            for i in range(num_enc):
                for e in range(2):
                    if i == 0:
                        inp = x_ref[e, t].reshape(NR, x_ref.shape[-1])
                    else:
                        inp = h_ref[e, i - 1].astype(jnp.bfloat16)
                    cell(e, i, inp)
            return carry

        lax.fori_loop(0, T, enc_step, 0)

        def dec_step(j, carry):
            for d in range(num_dec):
                layer = num_enc + d
                for e in range(2):
                    cell(e, layer, h_ref[e, layer - 1].astype(jnp.bfloat16))
            o_ref[0, j] = h_ref[0, L - 1][:, 0:OC8].astype(
                jnp.bfloat16).reshape(H, PW, OC8)
            o_ref[1, j] = h_ref[1, L - 1][:, 0:OC8].astype(
                jnp.bfloat16).reshape(H, PW, OC8)
            return carry

        lax.fori_loop(0, future_len, dec_step, 0)

    return body


def _pack(layers, CP):
    PW_taps = [(1, 1), (0, 0), (0, 1), (0, 2), (1, 0), (1, 2),
               (2, 0), (2, 1), (2, 2)]
    L = len(layers)
    C2 = 2 * CP
    w_all = jnp.zeros((L, 9 * C2, 4 * CP), jnp.float32)
    b_all = jnp.zeros((L, 1, 4 * CP), jnp.float32)
    for l, (w, b) in enumerate(layers):
        ctot, ch = w.shape[-2], w.shape[-1]
        cin = ctot - ch
        for gi, (ky, kx) in enumerate(PW_taps):
            wt = jnp.transpose(w[:, ky, kx], (1, 0, 2))
            wt = jnp.pad(wt, ((0, 0), (0, 0), (0, CP - ch)))
            wt = wt * jnp.array([0.5, 0.5, 0.5, 1.0]).reshape(1, 4, 1)
            wt = wt.reshape(ctot, 4 * CP)
            r0 = C2 * gi
            w_all = w_all.at[l, r0:r0 + cin, :].set(wt[:cin])
            w_all = w_all.at[l, r0 + CP:r0 + CP + ch, :].set(wt[cin:])
        bb = jnp.pad(b.reshape(4, ch), ((0, 0), (0, CP - ch)))
        bb = (bb * jnp.array([0.5, 0.5, 0.5, 1.0]).reshape(4, 1)).reshape(
            4 * CP)
        b_all = b_all.at[l, 0].set(bb)
    return w_all.astype(jnp.bfloat16), b_all


def _forward(x, enc_params, dec_params, future_len):
    B, c_in, T, H, W = x.shape
    num_enc, num_dec = len(enc_params), len(dec_params)
    L = num_enc + num_dec
    hidden = enc_params[0][0].shape[-1]
    c_out = dec_params[-1][0].shape[-1]
    CP = max(c_in, hidden, c_out)
    CP = ((CP + 31) // 32) * 32
    PW = W + 2
    NR = H * PW
    K9 = 9 * 2 * CP
    IMG0 = (-PW) % 16
    S = IMG0 + PW
    NQ = ((S + NR + PW + 1) + 15) // 16 * 16

    CIN8 = min(CP, ((c_in + 7) // 8) * 8)
    OC8 = min(CP, ((c_out + 7) // 8) * 8)

    w_all, b_all = _pack(list(enc_params) + list(dec_params), CP)

    x_l = jnp.transpose(x.astype(jnp.bfloat16), (0, 2, 3, 4, 1))
    x_l = jnp.pad(x_l, ((0, 0), (0, 0), (0, 0), (1, 1), (0, CIN8 - c_in)))

    body = _build_body(num_enc, num_dec, T, future_len, H, W, CP, S, NQ, OC8)

    out = pl.pallas_call(
        body,
        out_shape=jax.ShapeDtypeStruct((B, future_len, H, PW, OC8),
                                       jnp.bfloat16),
        grid=(B // 2,),
        in_specs=[
            pl.BlockSpec((2, T, H, PW, CIN8), lambda b: (b, 0, 0, 0, 0)),
            pl.BlockSpec((L, K9, 4 * CP), lambda b: (0, 0, 0)),
            pl.BlockSpec((L, 1, 4 * CP), lambda b: (0, 0, 0)),
        ],
        out_specs=pl.BlockSpec((2, future_len, H, PW, OC8),
                               lambda b: (b, 0, 0, 0, 0)),
        scratch_shapes=[
            pltpu.VMEM((2, L, NR, CP), jnp.float32),
            pltpu.VMEM((2, L, NR, CP), jnp.float32),
            pltpu.VMEM((2, NQ, K9), jnp.bfloat16),
        ],
        compiler_params=pltpu.CompilerParams(
            dimension_semantics=("parallel",),
            vmem_limit_bytes=64 * 1024 * 1024),
    )(x_l, w_all, b_all)

    return [jnp.transpose(out[:, j, :, 1:W + 1, :c_out], (0, 3, 1, 2))
            .astype(jnp.float32) for j in range(future_len)]


def kernel(x, w_e0, b_e0, w_e1, b_e1, w_d0, b_d0, w_d1, b_d1):
    enc_params = [(w_e0, b_e0), (w_e1, b_e1)]
    dec_params = [(w_d0, b_d0), (w_d1, b_d1)]
    return _forward(x, enc_params, dec_params, future_len=10)


# speedup vs baseline: 2.5058x; 2.1803x over previous
import functools

import jax
import jax.numpy as jnp
from jax import lax
from jax.experimental import pallas as pl
from jax.experimental.pallas import tpu as pltpu

_NE = 2


def _build_body(num_enc, num_dec, T, future_len, H, W, CP, OC8):
    L = num_enc + num_dec
    PW = W + 2
    NR = H * PW
    C2 = 2 * CP
    G4 = 4 * CP
    LW0 = 128
    LW1 = LW0 + NR
    deltas = [0, -PW - 1, -PW, -PW + 1, -1, 1, PW - 1, PW, PW + 1]

    def body(x_ref, w_ref, b_ref, o_ref, hb_ref, c_ref, q_ref):
        hb_ref[...] = jnp.zeros_like(hb_ref)
        c_ref[...] = jnp.zeros_like(c_ref)
        q_ref[...] = jnp.zeros_like(q_ref)

        row = lax.broadcasted_iota(jnp.int32, (G4, 1), 0)
        sig = row < 3 * CP
        sa = jnp.where(sig, 0.5, 1.0)
        sb = jnp.where(sig, 0.5, 0.0)
        ll = lax.broadcasted_iota(jnp.int32, (1, NR), 1) % PW
        lane_ok = jnp.logical_and(ll != 0, ll != PW - 1)

        def cell(e, layer, inp_bf16):
            wi = inp_bf16.shape[0]
            q_ref[e, 0:wi, LW0:LW1] = inp_bf16
            if wi < CP:
                q_ref[e, wi:CP, LW0:LW1] = jnp.zeros((CP - wi, NR),
                                                     jnp.bfloat16)
            q_ref[e, CP:C2, LW0:LW1] = hb_ref[e, layer]
            for gi in range(1, 9):
                d = deltas[gi]
                q_ref[e, C2 * gi:C2 * (gi + 1), LW0:LW1] = (
                    q_ref[e, 0:C2, LW0 + d:LW1 + d])
            gates = jnp.dot(w_ref[layer], q_ref[e, :, LW0:LW1],
                            preferred_element_type=jnp.float32) + b_ref[layer]
            act = jnp.tanh(gates) * sa + sb
            i_g = act[0 * CP:1 * CP]
            f_g = act[1 * CP:2 * CP]
            o_g = act[2 * CP:3 * CP]
            g_g = act[3 * CP:4 * CP]
            c_n = f_g * c_ref[e, layer] + i_g * g_g
            h_n = jnp.where(lane_ok, o_g * jnp.tanh(c_n), 0.0)
            c_ref[e, layer] = c_n
            hb_ref[e, layer] = h_n.astype(jnp.bfloat16)

        def enc_step(t, carry):
            for i in range(num_enc):
                for e in range(_NE):
                    inp = x_ref[e, t] if i == 0 else hb_ref[e, i - 1]
                    cell(e, i, inp)
            return carry

        lax.fori_loop(0, T, enc_step, 0)

        def dec_step(j, carry):
            for d in range(num_dec):
                layer = num_enc + d
                for e in range(_NE):
                    cell(e, layer, hb_ref[e, layer - 1])
            for e in range(_NE):
                o_ref[e, j] = hb_ref[e, L - 1][0:OC8, :]
            return carry

        lax.fori_loop(0, future_len, dec_step, 0)

    return body


def _pack(layers, CP):
    taps = [(1, 1), (0, 0), (0, 1), (0, 2), (1, 0), (1, 2),
            (2, 0), (2, 1), (2, 2)]
    L = len(layers)
    C2 = 2 * CP
    w_all = jnp.zeros((L, 9 * C2, 4 * CP), jnp.float32)
    b_all = jnp.zeros((L, 4 * CP, 1), jnp.float32)
    gate_s = jnp.array([0.5, 0.5, 0.5, 1.0])
    for l, (w, b) in enumerate(layers):
        ctot, ch = w.shape[-2], w.shape[-1]
        cin = ctot - ch
        for gi, (ky, kx) in enumerate(taps):
            wt = jnp.transpose(w[:, ky, kx], (1, 0, 2))
            wt = jnp.pad(wt, ((0, 0), (0, 0), (0, CP - ch)))
            wt = wt * gate_s.reshape(1, 4, 1)
            wt = wt.reshape(ctot, 4 * CP)
            r0 = C2 * gi
            w_all = w_all.at[l, r0:r0 + cin, :].set(wt[:cin])
            w_all = w_all.at[l, r0 + CP:r0 + CP + ch, :].set(wt[cin:])
        bb = jnp.pad(b.reshape(4, ch), ((0, 0), (0, CP - ch)))
        bb = (bb * gate_s.reshape(4, 1)).reshape(4 * CP)
        b_all = b_all.at[l, :, 0].set(bb)
    return jnp.transpose(w_all, (0, 2, 1)).astype(jnp.bfloat16), b_all


def _forward(x, enc_params, dec_params, future_len):
    B, c_in, T, H, W = x.shape
    num_enc, num_dec = len(enc_params), len(dec_params)
    L = num_enc + num_dec
    hidden = enc_params[0][0].shape[-1]
    c_out = dec_params[-1][0].shape[-1]
    CP = max(c_in, hidden, c_out)
    CP = ((CP + 31) // 32) * 32
    PW = W + 2
    NR = H * PW
    K9 = 9 * 2 * CP
    G4 = 4 * CP
    NL = ((128 + NR + PW + 2) + 127) // 128 * 128
    CIN8 = min(CP, ((c_in + 7) // 8) * 8)
    OC8 = min(CP, ((c_out + 7) // 8) * 8)

    w_all, b_all = _pack(list(enc_params) + list(dec_params), CP)

    x_l = jnp.transpose(x.astype(jnp.bfloat16), (0, 2, 1, 3, 4))
    x_l = jnp.pad(x_l, ((0, 0), (0, 0), (0, CIN8 - c_in), (0, 0), (1, 1)))
    x_l = x_l.reshape(B, T, CIN8, NR)

    body = _build_body(num_enc, num_dec, T, future_len, H, W, CP, OC8)

    out = pl.pallas_call(
        body,
        out_shape=jax.ShapeDtypeStruct((B, future_len, OC8, NR),
                                       jnp.bfloat16),
        grid=(B // _NE,),
        in_specs=[
            pl.BlockSpec((_NE, T, CIN8, NR), lambda b: (b, 0, 0, 0)),
            pl.BlockSpec((L, G4, K9), lambda b: (0, 0, 0)),
            pl.BlockSpec((L, G4, 1), lambda b: (0, 0, 0)),
        ],
        out_specs=pl.BlockSpec((_NE, future_len, OC8, NR),
                               lambda b: (b, 0, 0, 0)),
        scratch_shapes=[
            pltpu.VMEM((_NE, L, CP, NR), jnp.bfloat16),
            pltpu.VMEM((_NE, L, CP, NR), jnp.float32),
            pltpu.VMEM((_NE, K9, NL), jnp.bfloat16),
        ],
        compiler_params=pltpu.CompilerParams(
            dimension_semantics=("parallel",),
            vmem_limit_bytes=64 * 1024 * 1024),
    )(x_l, w_all, b_all)

    out = out.reshape(B, future_len, OC8, H, PW)
    return [out[:, j, :c_out, :, 1:W + 1].astype(jnp.float32)
            for j in range(future_len)]


def kernel(x, w_e0, b_e0, w_e1, b_e1, w_d0, b_d0, w_d1, b_d1):
    enc_params = [(w_e0, b_e0), (w_e1, b_e1)]
    dec_params = [(w_d0, b_d0), (w_d1, b_d1)]
    return _forward(x, enc_params, dec_params, future_len=10)


# speedup vs baseline: 2.5920x; 1.0344x over previous
import functools

import jax
import jax.numpy as jnp
from jax import lax
from jax.experimental import pallas as pl
from jax.experimental.pallas import tpu as pltpu

_NE = 4


def _build_body(num_enc, num_dec, T, future_len, H, W, CP, OC8):
    L = num_enc + num_dec
    PW = W + 2
    NR = H * PW
    C2 = 2 * CP
    G4 = 4 * CP
    LW0 = 128
    LW1 = LW0 + NR
    deltas = [0, -PW - 1, -PW, -PW + 1, -1, 1, PW - 1, PW, PW + 1]

    def body(x_ref, w_ref, b_ref, o_ref, hb_ref, c_ref, q_ref):
        hb_ref[...] = jnp.zeros_like(hb_ref)
        c_ref[...] = jnp.zeros_like(c_ref)
        q_ref[...] = jnp.zeros_like(q_ref)

        row = lax.broadcasted_iota(jnp.int32, (G4, 1), 0)
        sig = row < 3 * CP
        sa = jnp.where(sig, 0.5, 1.0)
        sb = jnp.where(sig, 0.5, 0.0)
        ll = lax.broadcasted_iota(jnp.int32, (1, NR), 1) % PW
        lane_ok = jnp.logical_and(ll != 0, ll != PW - 1)

        def cell(e, layer, inp_bf16):
            wi = inp_bf16.shape[0]
            q_ref[e, 0:wi, LW0:LW1] = inp_bf16
            if wi < CP:
                q_ref[e, wi:CP, LW0:LW1] = jnp.zeros((CP - wi, NR),
                                                     jnp.bfloat16)
            q_ref[e, CP:C2, LW0:LW1] = hb_ref[e, layer]
            for gi in range(1, 9):
                d = deltas[gi]
                q_ref[e, C2 * gi:C2 * (gi + 1), LW0:LW1] = (
                    q_ref[e, 0:C2, LW0 + d:LW1 + d])
            gates = jnp.dot(w_ref[layer], q_ref[e, :, LW0:LW1],
                            preferred_element_type=jnp.float32) + b_ref[layer]
            act = jnp.tanh(gates) * sa + sb
            i_g = act[0 * CP:1 * CP]
            f_g = act[1 * CP:2 * CP]
            o_g = act[2 * CP:3 * CP]
            g_g = act[3 * CP:4 * CP]
            c_n = f_g * c_ref[e, layer] + i_g * g_g
            h_n = jnp.where(lane_ok, o_g * jnp.tanh(c_n), 0.0)
            c_ref[e, layer] = c_n
            hb_ref[e, layer] = h_n.astype(jnp.bfloat16)

        def enc_step(t, carry):
            for i in range(num_enc):
                for e in range(_NE):
                    inp = x_ref[e, t] if i == 0 else hb_ref[e, i - 1]
                    cell(e, i, inp)
            return carry

        lax.fori_loop(0, T, enc_step, 0)

        def dec_step(j, carry):
            for d in range(num_dec):
                layer = num_enc + d
                for e in range(_NE):
                    cell(e, layer, hb_ref[e, layer - 1])
            for e in range(_NE):
                o_ref[e, j] = hb_ref[e, L - 1][0:OC8, :]
            return carry

        lax.fori_loop(0, future_len, dec_step, 0)

    return body


def _pack(layers, CP):
    taps = [(1, 1), (0, 0), (0, 1), (0, 2), (1, 0), (1, 2),
            (2, 0), (2, 1), (2, 2)]
    L = len(layers)
    C2 = 2 * CP
    w_all = jnp.zeros((L, 9 * C2, 4 * CP), jnp.float32)
    b_all = jnp.zeros((L, 4 * CP, 1), jnp.float32)
    gate_s = jnp.array([0.5, 0.5, 0.5, 1.0])
    for l, (w, b) in enumerate(layers):
        ctot, ch = w.shape[-2], w.shape[-1]
        cin = ctot - ch
        for gi, (ky, kx) in enumerate(taps):
            wt = jnp.transpose(w[:, ky, kx], (1, 0, 2))
            wt = jnp.pad(wt, ((0, 0), (0, 0), (0, CP - ch)))
            wt = wt * gate_s.reshape(1, 4, 1)
            wt = wt.reshape(ctot, 4 * CP)
            r0 = C2 * gi
            w_all = w_all.at[l, r0:r0 + cin, :].set(wt[:cin])
            w_all = w_all.at[l, r0 + CP:r0 + CP + ch, :].set(wt[cin:])
        bb = jnp.pad(b.reshape(4, ch), ((0, 0), (0, CP - ch)))
        bb = (bb * gate_s.reshape(4, 1)).reshape(4 * CP)
        b_all = b_all.at[l, :, 0].set(bb)
    return jnp.transpose(w_all, (0, 2, 1)).astype(jnp.bfloat16), b_all


def _forward(x, enc_params, dec_params, future_len):
    B, c_in, T, H, W = x.shape
    num_enc, num_dec = len(enc_params), len(dec_params)
    L = num_enc + num_dec
    hidden = enc_params[0][0].shape[-1]
    c_out = dec_params[-1][0].shape[-1]
    CP = max(c_in, hidden, c_out)
    CP = ((CP + 31) // 32) * 32
    PW = W + 2
    NR = H * PW
    K9 = 9 * 2 * CP
    G4 = 4 * CP
    NL = ((128 + NR + PW + 2) + 127) // 128 * 128
    CIN8 = min(CP, ((c_in + 7) // 8) * 8)
    OC8 = min(CP, ((c_out + 7) // 8) * 8)

    w_all, b_all = _pack(list(enc_params) + list(dec_params), CP)

    x_l = jnp.transpose(x.astype(jnp.bfloat16), (0, 2, 1, 3, 4))
    x_l = jnp.pad(x_l, ((0, 0), (0, 0), (0, CIN8 - c_in), (0, 0), (1, 1)))
    x_l = x_l.reshape(B, T, CIN8, NR)

    body = _build_body(num_enc, num_dec, T, future_len, H, W, CP, OC8)

    out = pl.pallas_call(
        body,
        out_shape=jax.ShapeDtypeStruct((B, future_len, OC8, NR),
                                       jnp.bfloat16),
        grid=(B // _NE,),
        in_specs=[
            pl.BlockSpec((_NE, T, CIN8, NR), lambda b: (b, 0, 0, 0)),
            pl.BlockSpec((L, G4, K9), lambda b: (0, 0, 0)),
            pl.BlockSpec((L, G4, 1), lambda b: (0, 0, 0)),
        ],
        out_specs=pl.BlockSpec((_NE, future_len, OC8, NR),
                               lambda b: (b, 0, 0, 0)),
        scratch_shapes=[
            pltpu.VMEM((_NE, L, CP, NR), jnp.bfloat16),
            pltpu.VMEM((_NE, L, CP, NR), jnp.float32),
            pltpu.VMEM((_NE, K9, NL), jnp.bfloat16),
        ],
        compiler_params=pltpu.CompilerParams(
            dimension_semantics=("parallel",),
            vmem_limit_bytes=64 * 1024 * 1024),
    )(x_l, w_all, b_all)

    out = out.reshape(B, future_len, OC8, H, PW)
    return [out[:, j, :c_out, :, 1:W + 1].astype(jnp.float32)
            for j in range(future_len)]


def kernel(x, w_e0, b_e0, w_e1, b_e1, w_d0, b_d0, w_d1, b_d1):
    enc_params = [(w_e0, b_e0), (w_e1, b_e1)]
    dec_params = [(w_d0, b_d0), (w_d1, b_d1)]
    return _forward(x, enc_params, dec_params, future_len=10)


# speedup vs baseline: 2.7827x; 1.0736x over previous
import functools

import jax
import jax.numpy as jnp
from jax import lax
from jax.experimental import pallas as pl
from jax.experimental.pallas import tpu as pltpu

_NE = 4


def _build_body(num_enc, num_dec, T, future_len, H, W, CP, OC8):
    L = num_enc + num_dec
    PW = W + 2
    NR = H * PW
    C2 = 2 * CP
    G4 = 4 * CP
    LW0 = 128
    LW1 = LW0 + NR
    deltas = [0, -PW - 1, -PW, -PW + 1, -1, 1, PW - 1, PW, PW + 1]

    def body(x_ref, w_ref, w0_ref, b_ref, o_ref, hb_ref, c_ref, q_ref):
        hb_ref[...] = jnp.zeros_like(hb_ref)
        c_ref[...] = jnp.zeros_like(c_ref)
        q_ref[...] = jnp.zeros_like(q_ref)

        row = lax.broadcasted_iota(jnp.int32, (G4, 1), 0)
        sig = row < 3 * CP
        sa = jnp.where(sig, 0.5, 1.0)
        sb = jnp.where(sig, 0.5, 0.0)
        ll = lax.broadcasted_iota(jnp.int32, (1, NR), 1) % PW
        lane_ok = jnp.logical_and(ll != 0, ll != PW - 1)

        def cell(e, layer, inp_bf16):
            wi = inp_bf16.shape[0]
            if layer == 0:
                gs = ((wi + CP + 15) // 16) * 16
                wref = w0_ref[0]
            else:
                gs = C2
                wref = w_ref[layer]
            q_ref[e, 0:wi, LW0:LW1] = inp_bf16
            q_ref[e, wi:wi + CP, LW0:LW1] = hb_ref[e, layer]
            for gi in range(1, 9):
                d = deltas[gi]
                q_ref[e, gs * gi:gs * (gi + 1), LW0:LW1] = (
                    q_ref[e, 0:gs, LW0 + d:LW1 + d])
            gates = jnp.dot(wref, q_ref[e, 0:9 * gs, LW0:LW1],
                            preferred_element_type=jnp.float32) + b_ref[layer]
            act = jnp.tanh(gates) * sa + sb
            i_g = act[0 * CP:1 * CP]
            f_g = act[1 * CP:2 * CP]
            o_g = act[2 * CP:3 * CP]
            g_g = act[3 * CP:4 * CP]
            c_n = f_g * c_ref[e, layer] + i_g * g_g
            h_n = jnp.where(lane_ok, o_g * jnp.tanh(c_n), 0.0)
            c_ref[e, layer] = c_n
            hb_ref[e, layer] = h_n.astype(jnp.bfloat16)

        def enc_step(t, carry):
            for i in range(num_enc):
                for e in range(_NE):
                    inp = x_ref[e, t] if i == 0 else hb_ref[e, i - 1]
                    cell(e, i, inp)
            return carry

        lax.fori_loop(0, T, enc_step, 0)

        def dec_step(j, carry):
            for d in range(num_dec):
                layer = num_enc + d
                for e in range(_NE):
                    cell(e, layer, hb_ref[e, layer - 1])
            for e in range(_NE):
                o_ref[e, j] = hb_ref[e, L - 1][0:OC8, :]
            return carry

        lax.fori_loop(0, future_len, dec_step, 0)

    return body


def _pack(layers, CP, gs0, cin8):
    taps = [(1, 1), (0, 0), (0, 1), (0, 2), (1, 0), (1, 2),
            (2, 0), (2, 1), (2, 2)]
    L = len(layers)
    C2 = 2 * CP
    w_all = jnp.zeros((L, 9 * C2, 4 * CP), jnp.float32)
    w0_all = jnp.zeros((1, 9 * gs0, 4 * CP), jnp.float32)
    b_all = jnp.zeros((L, 4 * CP, 1), jnp.float32)
    gate_s = jnp.array([0.5, 0.5, 0.5, 1.0])
    for l, (w, b) in enumerate(layers):
        ctot, ch = w.shape[-2], w.shape[-1]
        cin = ctot - ch
        for gi, (ky, kx) in enumerate(taps):
            wt = jnp.transpose(w[:, ky, kx], (1, 0, 2))
            wt = jnp.pad(wt, ((0, 0), (0, 0), (0, CP - ch)))
            wt = wt * gate_s.reshape(1, 4, 1)
            wt = wt.reshape(ctot, 4 * CP)
            r0 = C2 * gi
            w_all = w_all.at[l, r0:r0 + cin, :].set(wt[:cin])
            w_all = w_all.at[l, r0 + CP:r0 + CP + ch, :].set(wt[cin:])
            if l == 0:
                s0 = gs0 * gi
                w0_all = w0_all.at[0, s0:s0 + cin, :].set(wt[:cin])
                w0_all = w0_all.at[0, s0 + cin8:s0 + cin8 + ch, :].set(
                    wt[cin:])
        bb = jnp.pad(b.reshape(4, ch), ((0, 0), (0, CP - ch)))
        bb = (bb * gate_s.reshape(4, 1)).reshape(4 * CP)
        b_all = b_all.at[l, :, 0].set(bb)
    return (jnp.transpose(w_all, (0, 2, 1)).astype(jnp.bfloat16),
            jnp.transpose(w0_all, (0, 2, 1)).astype(jnp.bfloat16),
            b_all)


def _forward(x, enc_params, dec_params, future_len):
    B, c_in, T, H, W = x.shape
    num_enc, num_dec = len(enc_params), len(dec_params)
    L = num_enc + num_dec
    hidden = enc_params[0][0].shape[-1]
    c_out = dec_params[-1][0].shape[-1]
    CP = max(c_in, hidden, c_out)
    CP = ((CP + 31) // 32) * 32
    PW = W + 2
    NR = H * PW
    K9 = 9 * 2 * CP
    G4 = 4 * CP
    NL = ((128 + NR + PW + 2) + 127) // 128 * 128
    CIN8 = min(CP, ((c_in + 7) // 8) * 8)
    OC8 = min(CP, ((c_out + 7) // 8) * 8)

    GS0 = ((CIN8 + CP + 15) // 16) * 16
    w_all, w0_all, b_all = _pack(list(enc_params) + list(dec_params), CP,
                                 GS0, CIN8)

    x_l = jnp.transpose(x.astype(jnp.bfloat16), (0, 2, 1, 3, 4))
    x_l = jnp.pad(x_l, ((0, 0), (0, 0), (0, CIN8 - c_in), (0, 0), (1, 1)))
    x_l = x_l.reshape(B, T, CIN8, NR)

    body = _build_body(num_enc, num_dec, T, future_len, H, W, CP, OC8)

    out = pl.pallas_call(
        body,
        out_shape=jax.ShapeDtypeStruct((B, future_len, OC8, NR),
                                       jnp.bfloat16),
        grid=(B // _NE,),
        in_specs=[
            pl.BlockSpec((_NE, T, CIN8, NR), lambda b: (b, 0, 0, 0)),
            pl.BlockSpec((L, G4, K9), lambda b: (0, 0, 0)),
            pl.BlockSpec((1, G4, 9 * GS0), lambda b: (0, 0, 0)),
            pl.BlockSpec((L, G4, 1), lambda b: (0, 0, 0)),
        ],
        out_specs=pl.BlockSpec((_NE, future_len, OC8, NR),
                               lambda b: (b, 0, 0, 0)),
        scratch_shapes=[
            pltpu.VMEM((_NE, L, CP, NR), jnp.bfloat16),
            pltpu.VMEM((_NE, L, CP, NR), jnp.float32),
            pltpu.VMEM((_NE, K9, NL), jnp.bfloat16),
        ],
        compiler_params=pltpu.CompilerParams(
            dimension_semantics=("parallel",),
            vmem_limit_bytes=64 * 1024 * 1024),
    )(x_l, w_all, w0_all, b_all)

    out = out.reshape(B, future_len, OC8, H, PW)
    return [out[:, j, :c_out, :, 1:W + 1].astype(jnp.float32)
            for j in range(future_len)]


def kernel(x, w_e0, b_e0, w_e1, b_e1, w_d0, b_d0, w_d1, b_d1):
    enc_params = [(w_e0, b_e0), (w_e1, b_e1)]
    dec_params = [(w_d0, b_d0), (w_d1, b_d1)]
    return _forward(x, enc_params, dec_params, future_len=10)


# speedup vs baseline: 2.8924x; 1.0394x over previous
import functools

import jax
import jax.numpy as jnp
from jax import lax
from jax.experimental import pallas as pl
from jax.experimental.pallas import tpu as pltpu

_NE = 8


def _build_body(num_enc, num_dec, T, future_len, H, W, CP, OC8):
    L = num_enc + num_dec
    PW = W + 2
    NR = H * PW
    C2 = 2 * CP
    G4 = 4 * CP
    LW0 = 128
    LW1 = LW0 + NR
    deltas = [0, -PW - 1, -PW, -PW + 1, -1, 1, PW - 1, PW, PW + 1]

    def body(x_ref, w_ref, w0_ref, b_ref, o_ref, hb_ref, c_ref, q_ref):
        hb_ref[...] = jnp.zeros_like(hb_ref)
        c_ref[...] = jnp.zeros_like(c_ref)
        q_ref[...] = jnp.zeros_like(q_ref)

        row = lax.broadcasted_iota(jnp.int32, (G4, 1), 0)
        sig = row < 3 * CP
        sa = jnp.where(sig, 0.5, 1.0)
        sb = jnp.where(sig, 0.5, 0.0)
        ll = lax.broadcasted_iota(jnp.int32, (1, NR), 1) % PW
        lane_ok = jnp.logical_and(ll != 0, ll != PW - 1)

        def cell(e, layer, inp_bf16):
            wi = inp_bf16.shape[0]
            if layer == 0:
                gs = ((wi + CP + 15) // 16) * 16
                wref = w0_ref[0]
            else:
                gs = C2
                wref = w_ref[layer]
            q_ref[e, 0:wi, LW0:LW1] = inp_bf16
            q_ref[e, wi:wi + CP, LW0:LW1] = hb_ref[e, layer]
            for gi in range(1, 9):
                d = deltas[gi]
                q_ref[e, gs * gi:gs * (gi + 1), LW0:LW1] = (
                    q_ref[e, 0:gs, LW0 + d:LW1 + d])
            gates = jnp.dot(wref, q_ref[e, 0:9 * gs, LW0:LW1],
                            preferred_element_type=jnp.float32) + b_ref[layer]
            act = jnp.tanh(gates) * sa + sb
            i_g = act[0 * CP:1 * CP]
            f_g = act[1 * CP:2 * CP]
            o_g = act[2 * CP:3 * CP]
            g_g = act[3 * CP:4 * CP]
            c_n = f_g * c_ref[e, layer] + i_g * g_g
            h_n = jnp.where(lane_ok, o_g * jnp.tanh(c_n), 0.0)
            c_ref[e, layer] = c_n
            hb_ref[e, layer] = h_n.astype(jnp.bfloat16)

        def enc_step(t, carry):
            for i in range(num_enc):
                for e in range(_NE):
                    inp = x_ref[e, t] if i == 0 else hb_ref[e, i - 1]
                    cell(e, i, inp)
            return carry

        lax.fori_loop(0, T, enc_step, 0)

        def dec_step(j, carry):
            for d in range(num_dec):
                layer = num_enc + d
                for e in range(_NE):
                    cell(e, layer, hb_ref[e, layer - 1])
            for e in range(_NE):
                o_ref[e, j] = hb_ref[e, L - 1][0:OC8, :]
            return carry

        lax.fori_loop(0, future_len, dec_step, 0)

    return body


def _pack(layers, CP, gs0, cin8):
    taps = [(1, 1), (0, 0), (0, 1), (0, 2), (1, 0), (1, 2),
            (2, 0), (2, 1), (2, 2)]
    L = len(layers)
    C2 = 2 * CP
    w_all = jnp.zeros((L, 9 * C2, 4 * CP), jnp.float32)
    w0_all = jnp.zeros((1, 9 * gs0, 4 * CP), jnp.float32)
    b_all = jnp.zeros((L, 4 * CP, 1), jnp.float32)
    gate_s = jnp.array([0.5, 0.5, 0.5, 1.0])
    for l, (w, b) in enumerate(layers):
        ctot, ch = w.shape[-2], w.shape[-1]
        cin = ctot - ch
        for gi, (ky, kx) in enumerate(taps):
            wt = jnp.transpose(w[:, ky, kx], (1, 0, 2))
            wt = jnp.pad(wt, ((0, 0), (0, 0), (0, CP - ch)))
            wt = wt * gate_s.reshape(1, 4, 1)
            wt = wt.reshape(ctot, 4 * CP)
            r0 = C2 * gi
            w_all = w_all.at[l, r0:r0 + cin, :].set(wt[:cin])
            w_all = w_all.at[l, r0 + CP:r0 + CP + ch, :].set(wt[cin:])
            if l == 0:
                s0 = gs0 * gi
                w0_all = w0_all.at[0, s0:s0 + cin, :].set(wt[:cin])
                w0_all = w0_all.at[0, s0 + cin8:s0 + cin8 + ch, :].set(
                    wt[cin:])
        bb = jnp.pad(b.reshape(4, ch), ((0, 0), (0, CP - ch)))
        bb = (bb * gate_s.reshape(4, 1)).reshape(4 * CP)
        b_all = b_all.at[l, :, 0].set(bb)
    return (jnp.transpose(w_all, (0, 2, 1)).astype(jnp.bfloat16),
            jnp.transpose(w0_all, (0, 2, 1)).astype(jnp.bfloat16),
            b_all)


def _forward(x, enc_params, dec_params, future_len):
    B, c_in, T, H, W = x.shape
    num_enc, num_dec = len(enc_params), len(dec_params)
    L = num_enc + num_dec
    hidden = enc_params[0][0].shape[-1]
    c_out = dec_params[-1][0].shape[-1]
    CP = max(c_in, hidden, c_out)
    CP = ((CP + 31) // 32) * 32
    PW = W + 2
    NR = H * PW
    K9 = 9 * 2 * CP
    G4 = 4 * CP
    NL = ((128 + NR + PW + 2) + 127) // 128 * 128
    CIN8 = min(CP, ((c_in + 7) // 8) * 8)
    OC8 = min(CP, ((c_out + 7) // 8) * 8)

    GS0 = ((CIN8 + CP + 15) // 16) * 16
    w_all, w0_all, b_all = _pack(list(enc_params) + list(dec_params), CP,
                                 GS0, CIN8)

    x_l = jnp.transpose(x.astype(jnp.bfloat16), (0, 2, 1, 3, 4))
    x_l = jnp.pad(x_l, ((0, 0), (0, 0), (0, CIN8 - c_in), (0, 0), (1, 1)))
    x_l = x_l.reshape(B, T, CIN8, NR)

    body = _build_body(num_enc, num_dec, T, future_len, H, W, CP, OC8)

    out = pl.pallas_call(
        body,
        out_shape=jax.ShapeDtypeStruct((B, future_len, OC8, NR),
                                       jnp.bfloat16),
        grid=(B // _NE,),
        in_specs=[
            pl.BlockSpec((_NE, T, CIN8, NR), lambda b: (b, 0, 0, 0)),
            pl.BlockSpec((L, G4, K9), lambda b: (0, 0, 0)),
            pl.BlockSpec((1, G4, 9 * GS0), lambda b: (0, 0, 0)),
            pl.BlockSpec((L, G4, 1), lambda b: (0, 0, 0)),
        ],
        out_specs=pl.BlockSpec((_NE, future_len, OC8, NR),
                               lambda b: (b, 0, 0, 0)),
        scratch_shapes=[
            pltpu.VMEM((_NE, L, CP, NR), jnp.bfloat16),
            pltpu.VMEM((_NE, L, CP, NR), jnp.float32),
            pltpu.VMEM((_NE, K9, NL), jnp.bfloat16),
        ],
        compiler_params=pltpu.CompilerParams(
            dimension_semantics=("parallel",),
            vmem_limit_bytes=64 * 1024 * 1024),
    )(x_l, w_all, w0_all, b_all)

    out = out.reshape(B, future_len, OC8, H, PW)
    return [out[:, j, :c_out, :, 1:W + 1].astype(jnp.float32)
            for j in range(future_len)]


def kernel(x, w_e0, b_e0, w_e1, b_e1, w_d0, b_d0, w_d1, b_d1):
    enc_params = [(w_e0, b_e0), (w_e1, b_e1)]
    dec_params = [(w_d0, b_d0), (w_d1, b_d1)]
    return _forward(x, enc_params, dec_params, future_len=10)
